# baseline (speedup 1.0000x reference)
import functools

import jax
import jax.numpy as jnp
from jax import lax
from jax.experimental import pallas as pl
from jax.experimental.pallas import tpu as pltpu
from jax.experimental.pallas import tpu_sc as plsc

NC = 2
NS = 16
NW = NC * NS
CHUNK = 128


@functools.partial(jax.jit, static_argnums=(2, 3))
def _sc_embedding_lookup(idx2d, weight, n_chunks, embed_dim):
    total_chunks = idx2d.shape[0]
    n_rows = total_chunks * CHUNK
    mesh = plsc.VectorSubcoreMesh(core_axis_name="c", subcore_axis_name="s")

    @functools.partial(
        pl.kernel,
        out_type=jax.ShapeDtypeStruct((n_rows, embed_dim), jnp.float32),
        mesh=mesh,
        scratch_types=[
            pltpu.VMEM((n_chunks, CHUNK), jnp.int32),
            pltpu.VMEM((CHUNK, embed_dim), jnp.float32),
            pltpu.SemaphoreType.DMA,
        ],
        compiler_params=pltpu.CompilerParams(use_tc_tiling_on_sc=False),
    )
    def body(idx_hbm, table_hbm, out_hbm, idx_v, rows_v, sem):
        wid = lax.axis_index("s") * NC + lax.axis_index("c")
        chunk_base = wid * n_chunks
        row_base = chunk_base * CHUNK
        pltpu.sync_copy(idx_hbm.at[pl.ds(chunk_base, n_chunks)], idx_v)

        def step(j, carry):
            pltpu.async_copy(table_hbm.at[idx_v.at[j]], rows_v, sem).wait()
            pltpu.sync_copy(rows_v, out_hbm.at[pl.ds(row_base + j * CHUNK, CHUNK)])
            return carry

        lax.fori_loop(0, n_chunks, step, 0)

    return body(idx2d, weight)


def kernel(input_ids, weight):
    n_rows = input_ids.size
    embed_dim = weight.shape[1]
    total_chunks = n_rows // CHUNK
    n_chunks = total_chunks // NW
    idx2d = input_ids.reshape(total_chunks, CHUNK).astype(jnp.int32)
    out = _sc_embedding_lookup(idx2d, weight, n_chunks, embed_dim)
    return out.reshape(*input_ids.shape, embed_dim)


# speedup vs baseline: 1.1164x; 1.1164x over previous
import functools

import jax
import jax.numpy as jnp
from jax import lax
from jax.experimental import pallas as pl
from jax.experimental.pallas import tpu as pltpu
from jax.experimental.pallas import tpu_sc as plsc

NC = 2
NS = 16
NW = NC * NS
CHUNK = 128


NBUF = 8
PREF = 4


@functools.partial(jax.jit, static_argnums=(2, 3))
def _sc_embedding_lookup(idx2d, weight, n_chunks, embed_dim):
    total_chunks = idx2d.shape[0]
    n_rows = total_chunks * CHUNK
    mesh = plsc.VectorSubcoreMesh(core_axis_name="c", subcore_axis_name="s")

    @functools.partial(
        pl.kernel,
        out_type=jax.ShapeDtypeStruct((n_rows, embed_dim), jnp.float32),
        mesh=mesh,
        scratch_types=[
            pltpu.VMEM((n_chunks, CHUNK), jnp.int32),
            [pltpu.VMEM((CHUNK, embed_dim), jnp.float32) for _ in range(NBUF)],
            [pltpu.SemaphoreType.DMA for _ in range(NBUF)],
            [pltpu.SemaphoreType.DMA for _ in range(NBUF)],
        ],
        compiler_params=pltpu.CompilerParams(use_tc_tiling_on_sc=False),
    )
    def body(idx_hbm, table_hbm, out_hbm, idx_v, rows, sem_g, sem_p):
        wid = lax.axis_index("s") * NC + lax.axis_index("c")
        chunk_base = wid * n_chunks
        row_base = chunk_base * CHUNK
        pltpu.sync_copy(idx_hbm.at[pl.ds(chunk_base, n_chunks)], idx_v)

        def fire_gather(j, b):
            pltpu.async_copy(table_hbm.at[idx_v.at[j]], rows[b], sem_g[b])

        def wait_gather(b):
            pltpu.make_async_copy(
                table_hbm.at[pl.ds(0, CHUNK)], rows[b], sem_g[b]).wait()

        def fire_put(j, b):
            pltpu.async_copy(
                rows[b], out_hbm.at[pl.ds(row_base + j * CHUNK, CHUNK)], sem_p[b])

        def wait_put(b):
            pltpu.make_async_copy(
                rows[b], out_hbm.at[pl.ds(row_base, CHUNK)], sem_p[b]).wait()

        for b in range(PREF):
            fire_gather(b, b)

        def outer(i, carry):
            for b in range(NBUF):
                j = i * NBUF + b
                wait_gather(b)
                fire_put(j, b)
                bn = (b + PREF) % NBUF
                jn = j + PREF

                @pl.when(jn < n_chunks)
                def _():
                    @pl.when(j >= NBUF - PREF)
                    def _():
                        wait_put(bn)
                    fire_gather(jn, bn)
            return carry

        lax.fori_loop(0, n_chunks // NBUF, outer, 0)
        for b in range(NBUF):
            wait_put(b)

    return body(idx2d, weight)


def kernel(input_ids, weight):
    n_rows = input_ids.size
    embed_dim = weight.shape[1]
    total_chunks = n_rows // CHUNK
    n_chunks = total_chunks // NW
    idx2d = input_ids.reshape(total_chunks, CHUNK).astype(jnp.int32)
    out = _sc_embedding_lookup(idx2d, weight, n_chunks, embed_dim)
    return out.reshape(*input_ids.shape, embed_dim)
